# Initial kernel scaffold; baseline (speedup 1.0000x reference)
import jax
import jax.numpy as jnp
from jax.experimental import pallas as pl


def kernel(x):
    raise NotImplementedError("write your pallas kernel here")


# speedup vs baseline: 82.7100x; 82.7100x over previous
import jax
import jax.numpy as jnp
from jax import lax
from jax.experimental import pallas as pl
from jax.experimental.pallas import tpu as pltpu

_T = 2048
_B = 2
_C = 768
_TI = 256
_N = _B * _T * _C

_CE, _CRANK, _CDELTA, _CZP = 0, 1, 2, 3
_CP7, _CP8, _CP9 = 4, 5, 6
_CSA = 8
_CSF = 32


def _body(x_ref, o_ref, w_ref, m_ref):
    f32 = jnp.float32
    i32 = jnp.int32

    x = x_ref[...]
    gmax = jnp.max(x)
    gmin = jnp.min(x)
    s1 = jnp.sum(x)
    s2 = jnp.sum(x * x)
    var = (s2 - s1 * s1 / f32(_N)) / f32(_N - 1)
    std = jnp.sqrt(var)
    coeff = std / (gmax - gmin)

    def keyf(k):
        bits = jnp.where(k >= 0, k, k ^ i32(0x7FFFFFFF))
        return lax.bitcast_convert_type(bits, f32)

    k2 = _N // 2

    def mbody(_, lohi):
        lo, hi = lohi
        mid = (lo & hi) + ((lo ^ hi) >> 1)
        t = keyf(mid)
        cnt = jnp.sum((x_ref[...] <= t).astype(f32))
        p = cnt >= f32(k2 + 1)
        return (jnp.where(p, lo, mid + i32(1)), jnp.where(p, hi, mid))

    lo, _ = lax.fori_loop(0, 33, mbody, (i32(-2139095041), i32(2139095040)))
    v2 = keyf(lo)
    cstrict = jnp.sum((x < v2).astype(f32))
    vlow = jnp.max(jnp.where(x < v2, x, -jnp.inf))
    v1 = jnp.where(cstrict <= f32(k2 - 1), v2, vlow)
    mean = (v1 + v2) * f32(0.5)

    p_col = lax.broadcasted_iota(i32, (_T, 1), 0).astype(f32)
    j_row = lax.broadcasted_iota(i32, (1, _T), 1).astype(f32)
    i_col = lax.broadcasted_iota(i32, (_TI, 1), 0).astype(f32)
    c21_row = lax.broadcasted_iota(i32, (1, 21), 1).astype(f32)
    w3_row = lax.broadcasted_iota(i32, (1, 3), 1).astype(f32)
    t21_col = lax.broadcasted_iota(i32, (21, 1), 0).astype(f32)
    t_row = c21_row / f32(5.0) + (f32(511.0) / f32(254.0))
    k441 = lax.broadcasted_iota(i32, (1, 441), 1).astype(f32)
    cfloor441 = jnp.floor(k441 / f32(21.0))
    kmod441 = k441 - f32(21.0) * cfloor441

    def rowA(bi, carry):
        base = bi * _T
        xb = x_ref[pl.ds(base, _T), :]
        mn = jnp.min(xb, axis=1, keepdims=True)
        mx = jnp.max(xb, axis=1, keepdims=True)
        delta = (mx - mn) / f32(255.0)
        zp = jnp.round(-mn / delta)
        x_int = jnp.round(xb / delta) + zp
        e_col = (mx - mn) * jnp.where(p_col == f32(0.0), f32(1e8), f32(1.0))
        w_ref[pl.ds(base, _T), _CE:_CE + 1] = e_col
        w_ref[pl.ds(base, _T), _CDELTA:_CDELTA + 1] = delta
        w_ref[pl.ds(base, _T), _CZP:_CZP + 1] = zp

        def ewb(wi, e_mat):
            nlm1 = jnp.exp2(f32(7.0) + wi.astype(f32)) - f32(1.0)
            xi = x_int / nlm1
            xq = jnp.clip(xi, f32(0.0), f32(1.0)) * nlm1
            xd = (xq - zp) * delta
            ec = jnp.sum(jnp.abs(xb - xd), axis=1, keepdims=True)
            return e_mat + ec * (w3_row == wi.astype(f32))

        e_mat = lax.fori_loop(0, 3, ewb, jnp.zeros((_T, 3), f32))
        m_ref[pl.ds(bi, 1), 21:22] = jnp.sum(e_mat[:, 0:1]).reshape(1, 1)

        def cntb(ci, am_az):
            a_mat, az_row = am_az
            cif = ci.astype(f32)
            cval = cif / f32(5.0)
            band = (e_col * coeff) * cval
            chi = jnp.sum((xb < (mean + band)).astype(f32), axis=1, keepdims=True)
            clo = jnp.sum((xb < (mean - band)).astype(f32), axis=1, keepdims=True)
            clamp = jnp.where(ci > 0, f32(0.1) * f32(_C), f32(1e-6))
            a = jnp.maximum(chi - clo, clamp)
            a = f32(1.0) / a
            a = jnp.where(a > f32(1.0), f32(0.0), a)
            sel = (c21_row == cif).astype(f32)
            return (a_mat + a * sel, az_row + jnp.sum(jnp.abs(a)) * sel)

        a_mat, az_row = lax.fori_loop(
            0, 21, cntb, (jnp.zeros((_T, 21), f32), jnp.zeros((1, 21), f32)))
        m_ref[pl.ds(bi, 1), 0:21] = az_row

        def t2r(ti, acc):
            et = w_ref[pl.ds(base + ti * _TI, _TI), _CE:_CE + 1]
            m = (i_col + (ti * _TI).astype(f32)) == j_row
            return acc + jnp.sum(jnp.where(m, et, f32(0.0)), axis=0, keepdims=True)

        e_row = lax.fori_loop(0, _T // _TI, t2r, jnp.zeros((1, _T), f32))

        def rkb(ti, gle):
            gt, lt, eqb = gle
            et = w_ref[pl.ds(base + ti * _TI, _TI), _CE:_CE + 1]
            ii = i_col + (ti * _TI).astype(f32)
            gt = gt + jnp.sum((et > e_row).astype(f32), axis=0, keepdims=True)
            lt = lt + jnp.sum((et < e_row).astype(f32), axis=0, keepdims=True)
            eqm = (et == e_row) & (ii < j_row)
            eqb = eqb + jnp.sum(eqm.astype(f32), axis=0, keepdims=True)
            gtc = jnp.sum((e_row > et).astype(f32), axis=1, keepdims=True)
            eqc = jnp.sum(((e_row == et) & (j_row < ii)).astype(f32),
                          axis=1, keepdims=True)
            w_ref[pl.ds(base + ti * _TI, _TI), _CRANK:_CRANK + 1] = gtc + eqc
            return (gt, lt, eqb)

        z_row = jnp.zeros((1, _T), f32)
        gt, lt, eqb = lax.fori_loop(0, _T // _TI, rkb, (z_row, z_row, z_row))
        rank_row = gt + eqb
        q_row = lt + eqb

        v_asc = jnp.concatenate([e_col, a_mat], axis=1)

        def ohb(ti, carry2):
            rr = i_col + (ti * _TI).astype(f32)
            oha = (q_row == rr).astype(f32)
            ohf = (q_row == (f32(_T - 1) - rr)).astype(f32)
            ltr = (rank_row <= rr).astype(f32)
            w_ref[pl.ds(base + ti * _TI, _TI), _CSA:_CSA + 22] = jnp.dot(
                oha, v_asc, preferred_element_type=f32)
            w_ref[pl.ds(base + ti * _TI, _TI), _CSF:_CSF + 22] = jnp.dot(
                ohf, v_asc, preferred_element_type=f32)
            w_ref[pl.ds(base + ti * _TI, _TI), _CP7:_CP7 + 3] = jnp.dot(
                ltr, e_mat, preferred_element_type=f32)
            return carry2

        lax.fori_loop(0, _T // _TI, ohb, 0)
        return carry

    lax.fori_loop(0, _B, rowA, 0)

    m0 = m_ref[0:1, 0:21]
    m1 = m_ref[1:2, 0:21]

    def candb(ci, carry):
        errs, d0s, d1s = carry
        cif = ci.astype(f32)
        sel = (c21_row == cif).astype(f32)
        az = jnp.sum((m0 + m1) * sel) == f32(0.0)
        rc = ((cfloor441 == cif) & (kmod441 == t21_col)).astype(f32)
        e_ct = jnp.zeros((1, 21), f32)
        ds = []
        for b in range(_B):
            lo_r = b * _T
            hi_r = (b + 1) * _T
            ga = w_ref[lo_r:hi_r, _CSA:_CSA + 22]
            gf = w_ref[lo_r:hi_r, _CSF:_CSF + 22]
            t2 = ga[:, 0:1] * jnp.sum(ga[:, 1:22] * sel, axis=1, keepdims=True)
            t1 = gf[:, 0:1] * jnp.sum(gf[:, 1:22] * sel, axis=1, keepdims=True)
            s = (t1 - t2 * t_row) / std
            first = jnp.min(jnp.where(s < f32(0.0), p_col, f32(_T)),
                            axis=0, keepdims=True)
            mi = jnp.where(first >= f32(_T), f32(0.0), first)
            mi = jnp.where(az, f32(0.0), mi)
            d = jnp.maximum(mi - f32(1.0), f32(0.0))
            m2i = jnp.maximum(f32(_T - 2) - d, d)
            p7 = w_ref[lo_r:hi_r, _CP7:_CP7 + 1]
            p8 = w_ref[lo_r:hi_r, _CP8:_CP8 + 1]
            p9 = w_ref[lo_r:hi_r, _CP9:_CP9 + 1]
            g1 = jnp.sum(jnp.where(p_col == d, p9 - p8, f32(0.0)),
                         axis=0, keepdims=True)
            g2 = jnp.sum(jnp.where(p_col == m2i, p8 - p7, f32(0.0)),
                         axis=0, keepdims=True)
            e_ct = e_ct + g1 + g2 + m_ref[b:b + 1, 21:22]
            ds.append(d)
        errs = errs + jnp.dot(e_ct, rc, preferred_element_type=f32)
        d0s = d0s + jnp.dot(ds[0], rc, preferred_element_type=f32)
        d1s = d1s + jnp.dot(ds[1], rc, preferred_element_type=f32)
        return (errs, d0s, d1s)

    z441 = jnp.zeros((1, 441), f32)
    errs, d0s, d1s = lax.fori_loop(0, 21, candb, (z441, z441, z441))
    err = errs / f32(_N)
    min_err = jnp.min(err)
    last = jnp.max(jnp.where(err == min_err, k441, f32(-1.0)))
    lmask = (k441 == last).astype(f32)
    dbest0 = jnp.sum(lmask * d0s)
    dbest1 = jnp.sum(lmask * d1s)

    def rowC(bi, carry):
        base = bi * _T
        db = jnp.where(bi == 0, dbest0, dbest1)
        rank = w_ref[pl.ds(base, _T), _CRANK:_CRANK + 1]
        delta = w_ref[pl.ds(base, _T), _CDELTA:_CDELTA + 1]
        zp = w_ref[pl.ds(base, _T), _CZP:_CZP + 1]
        nl = jnp.where(rank <= db, f32(512.0),
                       jnp.where(rank <= f32(_T - 2) - db, f32(256.0),
                                 f32(128.0)))
        xb = x_ref[pl.ds(base, _T), :]
        x_int = jnp.round(xb / delta) + zp
        xi = x_int / (nl - f32(1.0))
        xq = jnp.clip(xi, f32(0.0), f32(1.0)) * (nl - f32(1.0))
        o_ref[pl.ds(base, _T), :] = (xq - zp) * delta
        return carry

    lax.fori_loop(0, _B, rowC, 0)


def kernel(x):
    B, T, C = x.shape
    x2d = x.reshape(B * T, C)
    y = pl.pallas_call(
        _body,
        out_shape=jax.ShapeDtypeStruct((B * T, C), jnp.float32),
        scratch_shapes=[
            pltpu.VMEM((B * T, 128), jnp.float32),
            pltpu.VMEM((8, 128), jnp.float32),
        ],
    )(x2d)
    return y.reshape(B, T, C)


# speedup vs baseline: 117.4569x; 1.4201x over previous
import jax
import jax.numpy as jnp
from jax import lax
from jax.experimental import pallas as pl
from jax.experimental.pallas import tpu as pltpu

_T = 2048
_B = 2
_C = 768
_TI = 256
_N = _B * _T * _C

_CE, _CRANK, _CDELTA, _CZP = 0, 1, 2, 3
_CP7, _CP8, _CP9 = 4, 5, 6
_CSA = 8
_CSF = 32


def _body(x_ref, o_ref, w_ref, m_ref):
    f32 = jnp.float32
    i32 = jnp.int32

    x = x_ref[...]
    gmax = jnp.max(x)
    gmin = jnp.min(x)
    s1 = jnp.sum(x)
    s2 = jnp.sum(x * x)
    var = (s2 - s1 * s1 / f32(_N)) / f32(_N - 1)
    std = jnp.sqrt(var)
    coeff = std / (gmax - gmin)

    def keyf(k):
        bits = jnp.where(k >= 0, k, k ^ i32(0x7FFFFFFF))
        return lax.bitcast_convert_type(bits, f32)

    k2 = _N // 2

    def mbody(_, lohi):
        lo, hi = lohi
        mid = (lo & hi) + ((lo ^ hi) >> 1)
        t = keyf(mid)
        cnt = jnp.sum((x_ref[...] <= t).astype(f32))
        p = cnt >= f32(k2 + 1)
        return (jnp.where(p, lo, mid + i32(1)), jnp.where(p, hi, mid))

    lo, _ = lax.fori_loop(0, 33, mbody, (i32(-2139095041), i32(2139095040)))
    v2 = keyf(lo)
    cstrict = jnp.sum((x < v2).astype(f32))
    vlow = jnp.max(jnp.where(x < v2, x, -jnp.inf))
    v1 = jnp.where(cstrict <= f32(k2 - 1), v2, vlow)
    mean = (v1 + v2) * f32(0.5)

    p_col = lax.broadcasted_iota(i32, (_T, 1), 0).astype(f32)
    j_row = lax.broadcasted_iota(i32, (1, _T), 1).astype(f32)
    i_col = lax.broadcasted_iota(i32, (_TI, 1), 0).astype(f32)
    c21_row = lax.broadcasted_iota(i32, (1, 21), 1).astype(f32)
    w3_row = lax.broadcasted_iota(i32, (1, 3), 1).astype(f32)
    t21_col = lax.broadcasted_iota(i32, (21, 1), 0).astype(f32)
    t_row = c21_row / f32(5.0) + (f32(511.0) / f32(254.0))
    k441 = lax.broadcasted_iota(i32, (1, 441), 1).astype(f32)
    cfloor441 = jnp.floor(k441 / f32(21.0))
    kmod441 = k441 - f32(21.0) * cfloor441

    def rowA(bi, carry):
        base = bi * _T
        xb = x_ref[pl.ds(base, _T), :]
        mn = jnp.min(xb, axis=1, keepdims=True)
        mx = jnp.max(xb, axis=1, keepdims=True)
        delta = (mx - mn) / f32(255.0)
        zp = jnp.round(-mn / delta)
        x_int = jnp.round(xb / delta) + zp
        e_col = (mx - mn) * jnp.where(p_col == f32(0.0), f32(1e8), f32(1.0))
        w_ref[pl.ds(base, _T), _CE:_CE + 1] = e_col
        w_ref[pl.ds(base, _T), _CDELTA:_CDELTA + 1] = delta
        w_ref[pl.ds(base, _T), _CZP:_CZP + 1] = zp

        def ewb(wi, e_mat):
            nlm1 = jnp.exp2(f32(7.0) + wi.astype(f32)) - f32(1.0)
            xi = x_int / nlm1
            xq = jnp.clip(xi, f32(0.0), f32(1.0)) * nlm1
            xd = (xq - zp) * delta
            ec = jnp.sum(jnp.abs(xb - xd), axis=1, keepdims=True)
            return e_mat + ec * (w3_row == wi.astype(f32))

        e_mat = lax.fori_loop(0, 3, ewb, jnp.zeros((_T, 3), f32))
        m_ref[pl.ds(bi, 1), 21:22] = jnp.sum(e_mat[:, 0:1]).reshape(1, 1)

        def cntb(ci, am_az):
            a_mat, az_row = am_az
            cif = ci.astype(f32)
            cval = cif / f32(5.0)
            band = (e_col * coeff) * cval
            cnt = jnp.sum((xb < (mean + band)).astype(f32)
                          - (xb < (mean - band)).astype(f32),
                          axis=1, keepdims=True)
            clamp = jnp.where(ci > 0, f32(0.1) * f32(_C), f32(1e-6))
            a = jnp.maximum(cnt, clamp)
            a = f32(1.0) / a
            a = jnp.where(a > f32(1.0), f32(0.0), a)
            sel = (c21_row == cif).astype(f32)
            return (a_mat + a * sel, az_row + jnp.sum(jnp.abs(a)) * sel)

        a_mat, az_row = lax.fori_loop(
            0, 21, cntb, (jnp.zeros((_T, 21), f32), jnp.zeros((1, 21), f32)))
        m_ref[pl.ds(bi, 1), 0:21] = az_row

        def t2r(ti, acc):
            et = w_ref[pl.ds(base + ti * _TI, _TI), _CE:_CE + 1]
            m = (i_col + (ti * _TI).astype(f32)) == j_row
            return acc + jnp.sum(jnp.where(m, et, f32(0.0)), axis=0, keepdims=True)

        e_row = lax.fori_loop(0, _T // _TI, t2r, jnp.zeros((1, _T), f32))

        def rkb(ti, gle):
            gt, lt, eqb = gle
            et = w_ref[pl.ds(base + ti * _TI, _TI), _CE:_CE + 1]
            ii = i_col + (ti * _TI).astype(f32)
            gt = gt + jnp.sum((et > e_row).astype(f32), axis=0, keepdims=True)
            lt = lt + jnp.sum((et < e_row).astype(f32), axis=0, keepdims=True)
            eqm = (et == e_row) & (ii < j_row)
            eqb = eqb + jnp.sum(eqm.astype(f32), axis=0, keepdims=True)
            gtc = jnp.sum((e_row > et).astype(f32), axis=1, keepdims=True)
            eqc = jnp.sum(((e_row == et) & (j_row < ii)).astype(f32),
                          axis=1, keepdims=True)
            w_ref[pl.ds(base + ti * _TI, _TI), _CRANK:_CRANK + 1] = gtc + eqc
            return (gt, lt, eqb)

        z_row = jnp.zeros((1, _T), f32)
        gt, lt, eqb = lax.fori_loop(0, _T // _TI, rkb, (z_row, z_row, z_row))
        rank_row = gt + eqb
        q_row = lt + eqb

        v_asc = jnp.concatenate([e_col, a_mat], axis=1)

        def ohb(ti, carry2):
            rr = i_col + (ti * _TI).astype(f32)
            oha = (q_row == rr).astype(f32)
            ohf = (q_row == (f32(_T - 1) - rr)).astype(f32)
            ltr = (rank_row <= rr).astype(f32)
            w_ref[pl.ds(base + ti * _TI, _TI), _CSA:_CSA + 22] = jnp.dot(
                oha, v_asc, preferred_element_type=f32)
            w_ref[pl.ds(base + ti * _TI, _TI), _CSF:_CSF + 22] = jnp.dot(
                ohf, v_asc, preferred_element_type=f32)
            w_ref[pl.ds(base + ti * _TI, _TI), _CP7:_CP7 + 3] = jnp.dot(
                ltr, e_mat, preferred_element_type=f32)
            return carry2

        lax.fori_loop(0, _T // _TI, ohb, 0)
        return carry

    lax.fori_loop(0, _B, rowA, 0)

    m0 = m_ref[0:1, 0:21]
    m1 = m_ref[1:2, 0:21]
    r_oh = (t21_col == cfloor441).astype(f32)
    ts_rep = kmod441 / f32(5.0) + (f32(511.0) / f32(254.0))
    az_mask = jnp.dot(m0 + m1, r_oh, preferred_element_type=f32) == f32(0.0)
    err = jnp.zeros((1, 441), f32)
    d_all = []
    for b in range(_B):
        lo_r = b * _T
        hi_r = (b + 1) * _T
        ga = w_ref[lo_r:hi_r, _CSA:_CSA + 22]
        gf = w_ref[lo_r:hi_r, _CSF:_CSF + 22]
        t2r = jnp.dot(ga[:, 0:1] * ga[:, 1:22], r_oh,
                      preferred_element_type=f32)
        t1r = jnp.dot(gf[:, 0:1] * gf[:, 1:22], r_oh,
                      preferred_element_type=f32)
        s = (t1r - t2r * ts_rep) / std
        first = jnp.min(jnp.where(s < f32(0.0), p_col, f32(_T)),
                        axis=0, keepdims=True)
        mi = jnp.where(first >= f32(_T), f32(0.0), first)
        mi = jnp.where(az_mask, f32(0.0), mi)
        d = jnp.maximum(mi - f32(1.0), f32(0.0))
        m2i = jnp.maximum(f32(_T - 2) - d, d)
        p7 = w_ref[lo_r:hi_r, _CP7:_CP7 + 1]
        p8 = w_ref[lo_r:hi_r, _CP8:_CP8 + 1]
        p9 = w_ref[lo_r:hi_r, _CP9:_CP9 + 1]
        g1 = jnp.sum(jnp.where(p_col == d, p9 - p8, f32(0.0)),
                     axis=0, keepdims=True)
        g2 = jnp.sum(jnp.where(p_col == m2i, p8 - p7, f32(0.0)),
                     axis=0, keepdims=True)
        err = err + g1 + g2 + m_ref[b:b + 1, 21:22]
        d_all.append(d)
    d0s, d1s = d_all
    err = err / f32(_N)
    min_err = jnp.min(err)
    last = jnp.max(jnp.where(err == min_err, k441, f32(-1.0)))
    lmask = (k441 == last).astype(f32)
    dbest0 = jnp.sum(lmask * d0s)
    dbest1 = jnp.sum(lmask * d1s)

    def rowC(bi, carry):
        base = bi * _T
        db = jnp.where(bi == 0, dbest0, dbest1)
        rank = w_ref[pl.ds(base, _T), _CRANK:_CRANK + 1]
        delta = w_ref[pl.ds(base, _T), _CDELTA:_CDELTA + 1]
        zp = w_ref[pl.ds(base, _T), _CZP:_CZP + 1]
        nl = jnp.where(rank <= db, f32(512.0),
                       jnp.where(rank <= f32(_T - 2) - db, f32(256.0),
                                 f32(128.0)))
        xb = x_ref[pl.ds(base, _T), :]
        x_int = jnp.round(xb / delta) + zp
        xi = x_int / (nl - f32(1.0))
        xq = jnp.clip(xi, f32(0.0), f32(1.0)) * (nl - f32(1.0))
        o_ref[pl.ds(base, _T), :] = (xq - zp) * delta
        return carry

    lax.fori_loop(0, _B, rowC, 0)


def kernel(x):
    B, T, C = x.shape
    x2d = x.reshape(B * T, C)
    y = pl.pallas_call(
        _body,
        out_shape=jax.ShapeDtypeStruct((B * T, C), jnp.float32),
        scratch_shapes=[
            pltpu.VMEM((B * T, 128), jnp.float32),
            pltpu.VMEM((8, 128), jnp.float32),
        ],
    )(x2d)
    return y.reshape(B, T, C)
